# baseline (speedup 1.0000x reference)
import functools

import jax
import jax.numpy as jnp
from jax import lax
from jax.experimental import pallas as pl
from jax.experimental.pallas import tpu as pltpu
from jax.experimental.pallas import tpu_sc as plsc

_N = 20000
_MAX_OUT = 128
_IOU = 0.7
_IM = 224.0
_NSC = 16
_NP = 20480
_PER = _NP // _NSC
_NV = _PER // 16
_NC = _NV // 16
_SELC = _MAX_OUT // 16

_mesh = plsc.VectorSubcoreMesh(core_axis_name="c", subcore_axis_name="s")


def _fill(x, dtype=jnp.float32):
    return jnp.full((16,), x, dtype)


def _sc_body(a_hbm, d_hbm, s_hbm, o_hbm, orec_hbm,
             av, dv, bv, volv, mscv, selv, resv, allv, rowv):
    cid = lax.axis_index("c")
    sid = lax.axis_index("s")

    @pl.when(cid == 0)
    def _():
        base = sid * _PER
        la = lax.broadcasted_iota(jnp.int32, (16,), 0)

        for c in range(6):
            pltpu.sync_copy(a_hbm.at[c, pl.ds(base, _PER)], av.at[c])
            pltpu.sync_copy(d_hbm.at[c, pl.ds(base, _PER)], dv.at[c])
        pltpu.sync_copy(s_hbm.at[pl.ds(base, _PER)], mscv)

        hi = _IM - 1.0

        def prep(i, cmax):
            sl = pl.ds(i * 16, 16)
            x1 = av[0, sl]
            y1 = av[1, sl]
            z1 = av[2, sl]
            x2 = av[3, sl]
            y2 = av[4, sl]
            z2 = av[5, sl]
            w = x2 - x1 + 1.0
            h = y2 - y1 + 1.0
            l = z2 - z1 + 1.0
            pcx = dv[0, sl] * w + (x1 + w * 0.5)
            pcy = dv[1, sl] * h + (y1 + h * 0.5)
            pcz = dv[2, sl] * l + (z1 + l * 0.5)
            pw = jnp.exp(dv[3, sl]) * w
            ph = jnp.exp(dv[4, sl]) * h
            pll = jnp.exp(dv[5, sl]) * l
            bx1 = jnp.clip(pcx - pw * 0.5, 0.0, hi)
            by1 = jnp.clip(pcy - ph * 0.5, 0.0, hi)
            bz1 = jnp.clip(pcz - pll * 0.5, 0.0, hi)
            bx2 = jnp.clip(pcx + pw * 0.5, 0.0, hi)
            by2 = jnp.clip(pcy + ph * 0.5, 0.0, hi)
            bz2 = jnp.clip(pcz + pll * 0.5, 0.0, hi)
            bv[0, sl] = bx1
            bv[1, sl] = by1
            bv[2, sl] = bz1
            bv[3, sl] = bx2
            bv[4, sl] = by2
            bv[5, sl] = bz2
            volv[sl] = (bx2 - bx1 + 1.0) * (by2 - by1 + 1.0) * (bz2 - bz1 + 1.0)
            cm = jnp.max(mscv[sl])
            lane_hit = la == lax.rem(i, 16)
            jj = i // 16
            return tuple(
                jnp.where(jnp.logical_and(lane_hit, jj == j), cm, cmax[j])
                for j in range(_NC))

        cmax0 = lax.fori_loop(0, _NV, prep,
                              tuple(_fill(-1.0) for _ in range(_NC)))

        for r in range(8):
            for j in range(_SELC):
                selv[r, pl.ds(j * 16, 16)] = jnp.zeros((16,), jnp.float32)

        def find(cmax):
            mf = cmax[0]
            for j in range(1, _NC):
                mf = jnp.maximum(mf, cmax[j])
            m = jnp.max(mf)
            cand = _fill(_NV, jnp.int32)
            for j in range(_NC):
                cand = jnp.minimum(
                    cand, jnp.where(cmax[j] == m, la + j * 16, _NV))
            cb = jnp.min(cand)
            chunk = mscv[pl.ds(cb * 16, 16)]
            lidx = jnp.min(jnp.where(chunk == m, cb * 16 + la, _PER - 1))
            return m, lidx, cb, chunk

        def check(lidx):
            offv = _fill(lidx, jnp.int32)
            cx1 = plsc.load_gather(bv, [_fill(0, jnp.int32), offv])
            cy1 = plsc.load_gather(bv, [_fill(1, jnp.int32), offv])
            cz1 = plsc.load_gather(bv, [_fill(2, jnp.int32), offv])
            cx2 = plsc.load_gather(bv, [_fill(3, jnp.int32), offv])
            cy2 = plsc.load_gather(bv, [_fill(4, jnp.int32), offv])
            cz2 = plsc.load_gather(bv, [_fill(5, jnp.int32), offv])
            cvol = plsc.load_gather(volv, [offv])
            acc = jnp.zeros((16,), jnp.float32)
            for j in range(_SELC):
                sl = pl.ds(j * 16, 16)
                xx1 = jnp.maximum(selv[0, sl], cx1)
                yy1 = jnp.maximum(selv[1, sl], cy1)
                zz1 = jnp.maximum(selv[2, sl], cz1)
                xx2 = jnp.minimum(selv[3, sl], cx2)
                yy2 = jnp.minimum(selv[4, sl], cy2)
                zz2 = jnp.minimum(selv[5, sl], cz2)
                inter = (jnp.maximum(xx2 - xx1 + 1.0, 0.0)
                         * jnp.maximum(yy2 - yy1 + 1.0, 0.0)
                         * jnp.maximum(zz2 - zz1 + 1.0, 0.0))
                iou = inter / (selv[6, sl] + cvol - inter)
                acc = jnp.maximum(acc, jnp.where(iou >= _IOU, selv[7, sl], 0.0))
            return jnp.max(acc)

        def knock_out(lidx, cb, chunk, cmax, on):
            hit = jnp.logical_and(on, cb * 16 + la == lidx)
            newchunk = jnp.where(hit, -1.0, chunk)
            mscv[pl.ds(cb * 16, 16)] = newchunk
            cm = jnp.max(newchunk)
            lane_hit = la == lax.rem(cb, 16)
            jj = cb // 16
            newcmax = tuple(
                jnp.where(jnp.logical_and(lane_hit, jj == j), cm, cmax[j])
                for j in range(_NC))
            return newcmax

        def it(t, carry):
            cmax = carry[:_NC]
            selcur = carry[_NC:]
            m0, l0, cb0, ch0 = find(cmax)
            s0 = check(l0)

            def wcond(wc):
                m, l, cb, ch, s = wc[:5]
                return jnp.logical_and(s > 0.0, m >= 0.0)

            def wbody(wc):
                m, l, cb, ch, s = wc[:5]
                cmx = wc[5:]
                cmx2 = knock_out(l, cb, ch, cmx, True)
                m2, l2, cb2, ch2 = find(cmx2)
                s2 = check(l2)
                return (m2, l2, cb2, ch2, s2) + cmx2

            fin = lax.while_loop(wcond, wbody, (m0, l0, cb0, ch0, s0) + cmax)
            m, lidx, cb, chunk = fin[0], fin[1], fin[2], fin[3]
            cmax = fin[5:]
            gidx = base + lidx

            ci = jnp.clip(la - 2, 0, 5)
            g = plsc.load_gather(bv, [ci, _fill(lidx, jnp.int32)])
            rec = jnp.where(la == 0, m,
                            jnp.where(la == 1, gidx.astype(jnp.float32), g))
            resv[...] = rec
            pltpu.sync_copy(resv, orec_hbm.at[sid])
            plsc.subcore_barrier()
            pltpu.sync_copy(orec_hbm, allv)
            plsc.subcore_barrier()

            mv = plsc.load_gather(allv, [la, _fill(0, jnp.int32)])
            bm = jnp.max(mv)
            wt = jnp.min(jnp.where(mv == bm, la, _NSC))
            best = plsc.load_gather(allv, [_fill(wt, jnp.int32), la])
            valid_f = jnp.where(bm >= 0.0, 1.0, 0.0)

            c0 = jnp.max(jnp.where(la == 2, best, -1e30))
            c1 = jnp.max(jnp.where(la == 3, best, -1e30))
            c2 = jnp.max(jnp.where(la == 4, best, -1e30))
            c3 = jnp.max(jnp.where(la == 5, best, -1e30))
            c4 = jnp.max(jnp.where(la == 6, best, -1e30))
            c5 = jnp.max(jnp.where(la == 7, best, -1e30))
            wvol = (c3 - c0 + 1.0) * (c4 - c1 + 1.0) * (c5 - c2 + 1.0)

            tl = lax.rem(t, 16)
            jch = t // 16
            hit = la == tl
            vals = (c0, c1, c2, c3, c4, c5, wvol, valid_f)
            selnew = tuple(jnp.where(hit, vals[r], selcur[r])
                           for r in range(8))
            for r in range(8):
                selv[r, pl.ds(jch * 16, 16)] = selnew[r]
            nxt = lax.rem(t + 1, 16) == 0
            selcur = tuple(jnp.where(nxt, 0.0, selnew[r]) for r in range(8))

            cmax = knock_out(lidx, cb, chunk, cmax, wt == sid)

            @pl.when(sid == 0)
            def _():
                row = jnp.zeros((16,), jnp.float32)
                for r, v in enumerate((c0, c1, c2, c3, c4, c5, bm)):
                    row = jnp.where(la == r, v, row)
                rowv[...] = row * valid_f
                pltpu.sync_copy(rowv, o_hbm.at[t])

            return cmax + selcur

        lax.fori_loop(0, _MAX_OUT, it,
                      cmax0 + tuple(_fill(0.0) for _ in range(8)))


_sc_nms = functools.partial(
    pl.kernel,
    out_type=[jax.ShapeDtypeStruct((_MAX_OUT, 16), jnp.float32),
              jax.ShapeDtypeStruct((_NSC, 16), jnp.float32)],
    mesh=_mesh,
    compiler_params=pltpu.CompilerParams(needs_layout_passes=False),
    scratch_types=[
        pltpu.VMEM((6, _PER), jnp.float32),
        pltpu.VMEM((6, _PER), jnp.float32),
        pltpu.VMEM((6, _PER), jnp.float32),
        pltpu.VMEM((_PER,), jnp.float32),
        pltpu.VMEM((_PER,), jnp.float32),
        pltpu.VMEM((8, _MAX_OUT), jnp.float32),
        pltpu.VMEM((16,), jnp.float32),
        pltpu.VMEM((_NSC, 16), jnp.float32),
        pltpu.VMEM((16,), jnp.float32),
    ],
)(_sc_body)


def kernel(anchors, rpn_bbox_pred, scores):
    pad = _NP - _N
    a = jnp.pad(anchors, ((0, pad), (0, 0))).T
    d = jnp.pad(rpn_bbox_pred, ((0, pad), (0, 0))).T
    s = jnp.pad(scores, (0, pad), constant_values=-1.0)
    out, _rec = _sc_nms(a, d, s)
    return out[:, :7]


# speedup vs baseline: 2.4855x; 2.4855x over previous
import functools

import jax
import jax.numpy as jnp
from jax import lax
from jax.experimental import pallas as pl
from jax.experimental.pallas import tpu as pltpu
from jax.experimental.pallas import tpu_sc as plsc

_N = 20000
_MAX_OUT = 128
_IOU = 0.7
_IM = 224.0
_NSC = 16
_NP = 20480
_PER = _NP // _NSC
_NV = _PER // 16
_NC = _NV // 16
_SELC = _MAX_OUT // 16

_mesh = plsc.VectorSubcoreMesh(core_axis_name="c", subcore_axis_name="s")


def _fill(x, dtype=jnp.float32):
    return jnp.full((16,), x, dtype)


def _sc_body(a_hbm, d_hbm, s_hbm, o_hbm,
             av, dv, bv, volv, mscv, selv, resv, allv, rowv, shared):
    cid = lax.axis_index("c")
    sid = lax.axis_index("s")

    @pl.when(cid == 0)
    def _():
        base = sid * _PER
        la = lax.broadcasted_iota(jnp.int32, (16,), 0)

        for c in range(6):
            pltpu.sync_copy(a_hbm.at[c, pl.ds(base, _PER)], av.at[c])
            pltpu.sync_copy(d_hbm.at[c, pl.ds(base, _PER)], dv.at[c])
        pltpu.sync_copy(s_hbm.at[pl.ds(base, _PER)], mscv)

        hi = _IM - 1.0

        def prep(i, cmax):
            sl = pl.ds(i * 16, 16)
            x1 = av[0, sl]
            y1 = av[1, sl]
            z1 = av[2, sl]
            x2 = av[3, sl]
            y2 = av[4, sl]
            z2 = av[5, sl]
            w = x2 - x1 + 1.0
            h = y2 - y1 + 1.0
            l = z2 - z1 + 1.0
            pcx = dv[0, sl] * w + (x1 + w * 0.5)
            pcy = dv[1, sl] * h + (y1 + h * 0.5)
            pcz = dv[2, sl] * l + (z1 + l * 0.5)
            pw = jnp.exp(dv[3, sl]) * w
            ph = jnp.exp(dv[4, sl]) * h
            pll = jnp.exp(dv[5, sl]) * l
            bx1 = jnp.clip(pcx - pw * 0.5, 0.0, hi)
            by1 = jnp.clip(pcy - ph * 0.5, 0.0, hi)
            bz1 = jnp.clip(pcz - pll * 0.5, 0.0, hi)
            bx2 = jnp.clip(pcx + pw * 0.5, 0.0, hi)
            by2 = jnp.clip(pcy + ph * 0.5, 0.0, hi)
            bz2 = jnp.clip(pcz + pll * 0.5, 0.0, hi)
            bv[0, sl] = bx1
            bv[1, sl] = by1
            bv[2, sl] = bz1
            bv[3, sl] = bx2
            bv[4, sl] = by2
            bv[5, sl] = bz2
            volv[sl] = (bx2 - bx1 + 1.0) * (by2 - by1 + 1.0) * (bz2 - bz1 + 1.0)
            cm = jnp.max(mscv[sl])
            lane_hit = la == lax.rem(i, 16)
            jj = i // 16
            return tuple(
                jnp.where(jnp.logical_and(lane_hit, jj == j), cm, cmax[j])
                for j in range(_NC))

        cmax0 = lax.fori_loop(0, _NV, prep,
                              tuple(_fill(-1.0) for _ in range(_NC)))

        for r in range(8):
            for j in range(_SELC):
                selv[r, pl.ds(j * 16, 16)] = jnp.zeros((16,), jnp.float32)

        def find(cmax):
            mf = cmax[0]
            for j in range(1, _NC):
                mf = jnp.maximum(mf, cmax[j])
            m = jnp.max(mf)
            cand = _fill(_NV, jnp.int32)
            for j in range(_NC):
                cand = jnp.minimum(
                    cand, jnp.where(cmax[j] == m, la + j * 16, _NV))
            cb = jnp.min(cand)
            chunk = mscv[pl.ds(cb * 16, 16)]
            lidx = jnp.min(jnp.where(chunk == m, cb * 16 + la, _PER - 1))
            return m, lidx, cb, chunk

        def check(lidx):
            offv = _fill(lidx, jnp.int32)
            cx1 = plsc.load_gather(bv, [_fill(0, jnp.int32), offv])
            cy1 = plsc.load_gather(bv, [_fill(1, jnp.int32), offv])
            cz1 = plsc.load_gather(bv, [_fill(2, jnp.int32), offv])
            cx2 = plsc.load_gather(bv, [_fill(3, jnp.int32), offv])
            cy2 = plsc.load_gather(bv, [_fill(4, jnp.int32), offv])
            cz2 = plsc.load_gather(bv, [_fill(5, jnp.int32), offv])
            cvol = plsc.load_gather(volv, [offv])
            acc = jnp.zeros((16,), jnp.float32)
            for j in range(_SELC):
                sl = pl.ds(j * 16, 16)
                xx1 = jnp.maximum(selv[0, sl], cx1)
                yy1 = jnp.maximum(selv[1, sl], cy1)
                zz1 = jnp.maximum(selv[2, sl], cz1)
                xx2 = jnp.minimum(selv[3, sl], cx2)
                yy2 = jnp.minimum(selv[4, sl], cy2)
                zz2 = jnp.minimum(selv[5, sl], cz2)
                inter = (jnp.maximum(xx2 - xx1 + 1.0, 0.0)
                         * jnp.maximum(yy2 - yy1 + 1.0, 0.0)
                         * jnp.maximum(zz2 - zz1 + 1.0, 0.0))
                iou = inter / (selv[6, sl] + cvol - inter)
                acc = jnp.maximum(acc, jnp.where(iou >= _IOU, selv[7, sl], 0.0))
            return jnp.max(acc)

        def knock_out(lidx, cb, chunk, cmax, on):
            hit = jnp.logical_and(on, cb * 16 + la == lidx)
            newchunk = jnp.where(hit, -1.0, chunk)
            mscv[pl.ds(cb * 16, 16)] = newchunk
            cm = jnp.max(newchunk)
            lane_hit = la == lax.rem(cb, 16)
            jj = cb // 16
            newcmax = tuple(
                jnp.where(jnp.logical_and(lane_hit, jj == j), cm, cmax[j])
                for j in range(_NC))
            return newcmax

        def it(t, carry):
            cmax = carry[:_NC]
            selcur = carry[_NC:]
            m0, l0, cb0, ch0 = find(cmax)
            s0 = check(l0)

            def wcond(wc):
                m, l, cb, ch, s = wc[:5]
                return jnp.logical_and(s > 0.0, m >= 0.0)

            def wbody(wc):
                m, l, cb, ch, s = wc[:5]
                cmx = wc[5:]
                cmx2 = knock_out(l, cb, ch, cmx, True)
                m2, l2, cb2, ch2 = find(cmx2)
                s2 = check(l2)
                return (m2, l2, cb2, ch2, s2) + cmx2

            fin = lax.while_loop(wcond, wbody, (m0, l0, cb0, ch0, s0) + cmax)
            m, lidx, cb, chunk = fin[0], fin[1], fin[2], fin[3]
            cmax = fin[5:]
            gidx = base + lidx

            ci = jnp.clip(la - 2, 0, 5)
            g = plsc.load_gather(bv, [ci, _fill(lidx, jnp.int32)])
            rec = jnp.where(la == 0, m,
                            jnp.where(la == 1, gidx.astype(jnp.float32), g))
            resv[...] = rec
            pltpu.sync_copy(resv, shared.at[sid])
            plsc.subcore_barrier()
            pltpu.sync_copy(shared.at[pl.ds(0, _NSC)], allv)
            plsc.subcore_barrier()

            mv = plsc.load_gather(allv, [la, _fill(0, jnp.int32)])
            bm = jnp.max(mv)
            wt = jnp.min(jnp.where(mv == bm, la, _NSC))
            best = plsc.load_gather(allv, [_fill(wt, jnp.int32), la])
            valid_f = jnp.where(bm >= 0.0, 1.0, 0.0)

            c0 = jnp.max(jnp.where(la == 2, best, -1e30))
            c1 = jnp.max(jnp.where(la == 3, best, -1e30))
            c2 = jnp.max(jnp.where(la == 4, best, -1e30))
            c3 = jnp.max(jnp.where(la == 5, best, -1e30))
            c4 = jnp.max(jnp.where(la == 6, best, -1e30))
            c5 = jnp.max(jnp.where(la == 7, best, -1e30))
            wvol = (c3 - c0 + 1.0) * (c4 - c1 + 1.0) * (c5 - c2 + 1.0)

            tl = lax.rem(t, 16)
            jch = t // 16
            hit = la == tl
            vals = (c0, c1, c2, c3, c4, c5, wvol, valid_f)
            selnew = tuple(jnp.where(hit, vals[r], selcur[r])
                           for r in range(8))
            for r in range(8):
                selv[r, pl.ds(jch * 16, 16)] = selnew[r]
            nxt = lax.rem(t + 1, 16) == 0
            selcur = tuple(jnp.where(nxt, 0.0, selnew[r]) for r in range(8))

            cmax = knock_out(lidx, cb, chunk, cmax, wt == sid)

            @pl.when(sid == 0)
            def _():
                row = jnp.zeros((16,), jnp.float32)
                for r, v in enumerate((c0, c1, c2, c3, c4, c5, bm)):
                    row = jnp.where(la == r, v, row)
                rowv[...] = row * valid_f
                pltpu.sync_copy(rowv, o_hbm.at[t])

            return cmax + selcur

        lax.fori_loop(0, _MAX_OUT, it,
                      cmax0 + tuple(_fill(0.0) for _ in range(8)))


_sc_nms = functools.partial(
    pl.kernel,
    out_type=jax.ShapeDtypeStruct((_MAX_OUT, 16), jnp.float32),
    mesh=_mesh,
    compiler_params=pltpu.CompilerParams(needs_layout_passes=False),
    scratch_types=[
        pltpu.VMEM((6, _PER), jnp.float32),
        pltpu.VMEM((6, _PER), jnp.float32),
        pltpu.VMEM((6, _PER), jnp.float32),
        pltpu.VMEM((_PER,), jnp.float32),
        pltpu.VMEM((_PER,), jnp.float32),
        pltpu.VMEM((8, _MAX_OUT), jnp.float32),
        pltpu.VMEM((16,), jnp.float32),
        pltpu.VMEM((_NSC, 16), jnp.float32),
        pltpu.VMEM((16,), jnp.float32),
        pltpu.VMEM_SHARED((128, 16), jnp.float32),
    ],
)(_sc_body)


def kernel(anchors, rpn_bbox_pred, scores):
    pad = _NP - _N
    a = jnp.pad(anchors, ((0, pad), (0, 0))).T
    d = jnp.pad(rpn_bbox_pred, ((0, pad), (0, 0))).T
    s = jnp.pad(scores, (0, pad), constant_values=-1.0)
    out = _sc_nms(a, d, s)
    return out[:, :7]


# speedup vs baseline: 2.6164x; 1.0527x over previous
import functools

import jax
import jax.numpy as jnp
from jax import lax
from jax.experimental import pallas as pl
from jax.experimental.pallas import tpu as pltpu
from jax.experimental.pallas import tpu_sc as plsc

_N = 20000
_MAX_OUT = 128
_IOU = 0.7
_IM = 224.0
_NSC = 16
_NP = 20480
_PER = _NP // _NSC
_NV = _PER // 16
_NC = _NV // 16
_SELC = _MAX_OUT // 16

_mesh = plsc.VectorSubcoreMesh(core_axis_name="c", subcore_axis_name="s")


def _fill(x, dtype=jnp.float32):
    return jnp.full((16,), x, dtype)


def _sc_body(a_hbm, d_hbm, s_hbm, o_hbm,
             av, dv, bv, volv, mscv, selv, resv, allv, rowv, shared):
    cid = lax.axis_index("c")
    sid = lax.axis_index("s")

    @pl.when(cid == 0)
    def _():
        base = sid * _PER
        la = lax.broadcasted_iota(jnp.int32, (16,), 0)

        for c in range(6):
            pltpu.sync_copy(a_hbm.at[c, pl.ds(base, _PER)], av.at[c])
            pltpu.sync_copy(d_hbm.at[c, pl.ds(base, _PER)], dv.at[c])
        pltpu.sync_copy(s_hbm.at[pl.ds(base, _PER)], mscv)

        hi = _IM - 1.0

        def prep(i, cmax):
            sl = pl.ds(i * 16, 16)
            x1 = av[0, sl]
            y1 = av[1, sl]
            z1 = av[2, sl]
            x2 = av[3, sl]
            y2 = av[4, sl]
            z2 = av[5, sl]
            w = x2 - x1 + 1.0
            h = y2 - y1 + 1.0
            l = z2 - z1 + 1.0
            pcx = dv[0, sl] * w + (x1 + w * 0.5)
            pcy = dv[1, sl] * h + (y1 + h * 0.5)
            pcz = dv[2, sl] * l + (z1 + l * 0.5)
            pw = jnp.exp(dv[3, sl]) * w
            ph = jnp.exp(dv[4, sl]) * h
            pll = jnp.exp(dv[5, sl]) * l
            bx1 = jnp.clip(pcx - pw * 0.5, 0.0, hi)
            by1 = jnp.clip(pcy - ph * 0.5, 0.0, hi)
            bz1 = jnp.clip(pcz - pll * 0.5, 0.0, hi)
            bx2 = jnp.clip(pcx + pw * 0.5, 0.0, hi)
            by2 = jnp.clip(pcy + ph * 0.5, 0.0, hi)
            bz2 = jnp.clip(pcz + pll * 0.5, 0.0, hi)
            bv[0, sl] = bx1
            bv[1, sl] = by1
            bv[2, sl] = bz1
            bv[3, sl] = bx2
            bv[4, sl] = by2
            bv[5, sl] = bz2
            volv[sl] = (bx2 - bx1 + 1.0) * (by2 - by1 + 1.0) * (bz2 - bz1 + 1.0)
            cm = jnp.max(mscv[sl])
            lane_hit = la == lax.rem(i, 16)
            jj = i // 16
            return tuple(
                jnp.where(jnp.logical_and(lane_hit, jj == j), cm, cmax[j])
                for j in range(_NC))

        cmax0 = lax.fori_loop(0, _NV, prep,
                              tuple(_fill(-1.0) for _ in range(_NC)))

        for r in range(8):
            for j in range(_SELC):
                selv[r, pl.ds(j * 16, 16)] = jnp.zeros((16,), jnp.float32)

        def find(cmax):
            mf = cmax[0]
            for j in range(1, _NC):
                mf = jnp.maximum(mf, cmax[j])
            m = jnp.max(mf)
            cand = _fill(_NV, jnp.int32)
            for j in range(_NC):
                cand = jnp.minimum(
                    cand, jnp.where(cmax[j] == m, la + j * 16, _NV))
            cb = jnp.min(cand)
            chunk = mscv[pl.ds(cb * 16, 16)]
            lidx = jnp.min(jnp.where(chunk == m, cb * 16 + la, _PER - 1))
            return m, lidx, cb, chunk

        def check(lidx):
            offv = _fill(lidx, jnp.int32)
            cx1 = plsc.load_gather(bv, [_fill(0, jnp.int32), offv])
            cy1 = plsc.load_gather(bv, [_fill(1, jnp.int32), offv])
            cz1 = plsc.load_gather(bv, [_fill(2, jnp.int32), offv])
            cx2 = plsc.load_gather(bv, [_fill(3, jnp.int32), offv])
            cy2 = plsc.load_gather(bv, [_fill(4, jnp.int32), offv])
            cz2 = plsc.load_gather(bv, [_fill(5, jnp.int32), offv])
            cvol = plsc.load_gather(volv, [offv])
            acc = jnp.zeros((16,), jnp.float32)
            for j in range(_SELC):
                sl = pl.ds(j * 16, 16)
                xx1 = jnp.maximum(selv[0, sl], cx1)
                yy1 = jnp.maximum(selv[1, sl], cy1)
                zz1 = jnp.maximum(selv[2, sl], cz1)
                xx2 = jnp.minimum(selv[3, sl], cx2)
                yy2 = jnp.minimum(selv[4, sl], cy2)
                zz2 = jnp.minimum(selv[5, sl], cz2)
                inter = (jnp.maximum(xx2 - xx1 + 1.0, 0.0)
                         * jnp.maximum(yy2 - yy1 + 1.0, 0.0)
                         * jnp.maximum(zz2 - zz1 + 1.0, 0.0))
                iou = inter / (selv[6, sl] + cvol - inter)
                acc = jnp.maximum(acc, jnp.where(iou >= _IOU, selv[7, sl], 0.0))
            return jnp.max(acc)

        def knock_out(lidx, cb, chunk, cmax, on):
            hit = jnp.logical_and(on, cb * 16 + la == lidx)
            newchunk = jnp.where(hit, -1.0, chunk)
            mscv[pl.ds(cb * 16, 16)] = newchunk
            cm = jnp.max(newchunk)
            lane_hit = la == lax.rem(cb, 16)
            jj = cb // 16
            newcmax = tuple(
                jnp.where(jnp.logical_and(lane_hit, jj == j), cm, cmax[j])
                for j in range(_NC))
            return newcmax

        def it(t, carry):
            cmax = carry[:_NC]
            selcur = carry[_NC:]
            m0, l0, cb0, ch0 = find(cmax)
            s0 = check(l0)

            def wcond(wc):
                m, l, cb, ch, s = wc[:5]
                return jnp.logical_and(s > 0.0, m >= 0.0)

            def wbody(wc):
                m, l, cb, ch, s = wc[:5]
                cmx = wc[5:]
                cmx2 = knock_out(l, cb, ch, cmx, True)
                m2, l2, cb2, ch2 = find(cmx2)
                s2 = check(l2)
                return (m2, l2, cb2, ch2, s2) + cmx2

            fin = lax.while_loop(wcond, wbody, (m0, l0, cb0, ch0, s0) + cmax)
            m, lidx, cb, chunk = fin[0], fin[1], fin[2], fin[3]
            cmax = fin[5:]
            gidx = base + lidx

            ci = jnp.clip(la - 2, 0, 5)
            g = plsc.load_gather(bv, [ci, _fill(lidx, jnp.int32)])
            rec = jnp.where(la == 0, m,
                            jnp.where(la == 1, gidx.astype(jnp.float32), g))
            resv[...] = rec
            srow = sid + lax.rem(t, 2) * 32
            pltpu.sync_copy(resv, shared.at[srow])
            plsc.subcore_barrier()
            pltpu.sync_copy(shared.at[pl.ds(lax.rem(t, 2) * 32, _NSC)], allv)

            mv = plsc.load_gather(allv, [la, _fill(0, jnp.int32)])
            bm = jnp.max(mv)
            wt = jnp.min(jnp.where(mv == bm, la, _NSC))
            valid_f = jnp.where(bm >= 0.0, 1.0, 0.0)

            wtv = _fill(wt, jnp.int32)
            c0 = plsc.load_gather(allv, [wtv, _fill(2, jnp.int32)])
            c1 = plsc.load_gather(allv, [wtv, _fill(3, jnp.int32)])
            c2 = plsc.load_gather(allv, [wtv, _fill(4, jnp.int32)])
            c3 = plsc.load_gather(allv, [wtv, _fill(5, jnp.int32)])
            c4 = plsc.load_gather(allv, [wtv, _fill(6, jnp.int32)])
            c5 = plsc.load_gather(allv, [wtv, _fill(7, jnp.int32)])
            wvol = (c3 - c0 + 1.0) * (c4 - c1 + 1.0) * (c5 - c2 + 1.0)

            tl = lax.rem(t, 16)
            jch = t // 16
            hit = la == tl
            vals = (c0, c1, c2, c3, c4, c5, wvol, valid_f)
            selnew = tuple(jnp.where(hit, vals[r], selcur[r])
                           for r in range(8))
            for r in range(8):
                selv[r, pl.ds(jch * 16, 16)] = selnew[r]
            nxt = lax.rem(t + 1, 16) == 0
            selcur = tuple(jnp.where(nxt, 0.0, selnew[r]) for r in range(8))

            cmax = knock_out(lidx, cb, chunk, cmax, wt == sid)

            @pl.when(sid == 0)
            def _():
                row = jnp.zeros((16,), jnp.float32)
                for r, v in enumerate((c0, c1, c2, c3, c4, c5, bm)):
                    row = jnp.where(la == r, v, row)
                rowv[...] = row * valid_f
                pltpu.sync_copy(rowv, o_hbm.at[t])

            return cmax + selcur

        lax.fori_loop(0, _MAX_OUT, it,
                      cmax0 + tuple(_fill(0.0) for _ in range(8)))


_sc_nms = functools.partial(
    pl.kernel,
    out_type=jax.ShapeDtypeStruct((_MAX_OUT, 16), jnp.float32),
    mesh=_mesh,
    compiler_params=pltpu.CompilerParams(needs_layout_passes=False),
    scratch_types=[
        pltpu.VMEM((6, _PER), jnp.float32),
        pltpu.VMEM((6, _PER), jnp.float32),
        pltpu.VMEM((6, _PER), jnp.float32),
        pltpu.VMEM((_PER,), jnp.float32),
        pltpu.VMEM((_PER,), jnp.float32),
        pltpu.VMEM((8, _MAX_OUT), jnp.float32),
        pltpu.VMEM((16,), jnp.float32),
        pltpu.VMEM((_NSC, 16), jnp.float32),
        pltpu.VMEM((16,), jnp.float32),
        pltpu.VMEM_SHARED((128, 16), jnp.float32),
    ],
)(_sc_body)


def kernel(anchors, rpn_bbox_pred, scores):
    pad = _NP - _N
    a = jnp.pad(anchors, ((0, pad), (0, 0))).T
    d = jnp.pad(rpn_bbox_pred, ((0, pad), (0, 0))).T
    s = jnp.pad(scores, (0, pad), constant_values=-1.0)
    out = _sc_nms(a, d, s)
    return out[:, :7]
